# Initial kernel scaffold; baseline (speedup 1.0000x reference)
import functools

import jax
import jax.numpy as jnp
import numpy as np
from jax import lax
from jax.experimental import pallas as pl
from jax.experimental.pallas import tpu as pltpu
from jax.experimental.pallas import tpu_sc as plsc

_N = 10000
_E = 640000
_HID = 64
_NH = 8
_NC = 2
_NS = 16
_NW = _NC * _NS
_EPT = _E // _NW
_CHUNK = 80
_NCHUNK = _EPT // _CHUNK
_RPT = _N // _NS

_mesh = plsc.VectorSubcoreMesh(core_axis_name="c", subcore_axis_name="s")


@functools.partial(
    pl.kernel,
    mesh=_mesh,
    out_type=jax.ShapeDtypeStruct((_NC, _N, 16), jnp.float32),
    scratch_types=[
        pltpu.VMEM((_CHUNK,), jnp.int32),
        pltpu.VMEM((_CHUNK, 16), jnp.float32),
        pltpu.VMEM_SHARED((_N, 16), jnp.float32),
    ],
)
def _sc_degree(ei_hbm, ones_hbm, zeros_hbm, out_hbm, dstv, ones_v, deg_s):
    c = lax.axis_index("c")
    s = lax.axis_index("s")
    pltpu.sync_copy(zeros_hbm.at[pl.ds(s * _RPT, _RPT)],
                    deg_s.at[pl.ds(s * _RPT, _RPT)])
    pltpu.sync_copy(ones_hbm, ones_v)
    plsc.subcore_barrier()
    base0 = (c * _NS + s) * _EPT

    def body(j, carry):
        b = pl.multiple_of(base0 + j * _CHUNK, 8)
        pltpu.sync_copy(ei_hbm.at[1, pl.ds(b, _CHUNK)], dstv)
        pltpu.sync_copy(ones_v, deg_s.at[dstv], add=True)
        return carry

    lax.fori_loop(0, _NCHUNK, body, 0)
    plsc.subcore_barrier()
    pltpu.sync_copy(deg_s.at[pl.ds(s * _RPT, _RPT)],
                    out_hbm.at[c, pl.ds(s * _RPT, _RPT)])


def _make_edge_scatter(width):
    @functools.partial(
        pl.kernel,
        mesh=_mesh,
        out_type=jax.ShapeDtypeStruct((_NC, _N, width), jnp.float32),
        scratch_types=[
            pltpu.VMEM((_CHUNK,), jnp.int32),
            pltpu.VMEM((_CHUNK,), jnp.int32),
            pltpu.VMEM((_CHUNK, width), jnp.float32),
            pltpu.VMEM_SHARED((_N, width), jnp.float32),
            pltpu.SemaphoreType.DMA,
        ],
    )
    def scat(tab_hbm, ei_hbm, zeros_hbm, out_hbm, srcv, dstv, rows, agg_s, sem):
        c = lax.axis_index("c")
        s = lax.axis_index("s")
        pltpu.sync_copy(zeros_hbm.at[pl.ds(s * _RPT, _RPT)],
                        agg_s.at[pl.ds(s * _RPT, _RPT)])
        plsc.subcore_barrier()
        base0 = (c * _NS + s) * _EPT

        def body(j, carry):
            b = pl.multiple_of(base0 + j * _CHUNK, 8)
            pltpu.sync_copy(ei_hbm.at[0, pl.ds(b, _CHUNK)], srcv)
            pltpu.sync_copy(ei_hbm.at[1, pl.ds(b, _CHUNK)], dstv)
            pltpu.async_copy(tab_hbm.at[srcv], rows, sem).wait()
            pltpu.sync_copy(rows, agg_s.at[dstv], add=True)
            return carry

        lax.fori_loop(0, _NCHUNK, body, 0)
        plsc.subcore_barrier()
        pltpu.sync_copy(agg_s.at[pl.ds(s * _RPT, _RPT)],
                        out_hbm.at[c, pl.ds(s * _RPT, _RPT)])

    return scat


_sc_scatter16 = _make_edge_scatter(16)
_sc_scatter64 = _make_edge_scatter(_HID)


def _prep_body(deg2_ref, x16_ref, xp_ref, dis_ref):
    d = deg2_ref[...]
    deg = d[0, :, 0:1] + d[1, :, 0:1] + 1.0
    dis = lax.rsqrt(deg)
    dis_ref[...] = dis
    xp_ref[...] = x16_ref[...] * dis


def _bn_relu(t, g, be):
    m = jnp.mean(t, axis=0, keepdims=True)
    v = jnp.mean((t - m) ** 2, axis=0, keepdims=True)
    h = (t - m) * lax.rsqrt(v + 1e-5) * g + be
    return jnp.maximum(h, 0.0)


def _layer0_body(agg_ref, xp_ref, dis_ref, W0_ref, b_ref, g_ref, be_ref,
                 W1_ref, out_ref):
    a = agg_ref[...]
    dis = dis_ref[...]
    t16 = a[0] + a[1] + xp_ref[...]
    t = dis * lax.dot_general(t16, W0_ref[...], (((1,), (0,)), ((), ())),
                              preferred_element_type=jnp.float32) + b_ref[...]
    h = _bn_relu(t, g_ref[...], be_ref[...])
    out_ref[...] = dis * lax.dot_general(
        h, W1_ref[...], (((1,), (0,)), ((), ())),
        preferred_element_type=jnp.float32)


def _layer_body(agg_ref, hp_ref, dis_ref, b_ref, g_ref, be_ref, W_ref,
                out_ref):
    a = agg_ref[...]
    dis = dis_ref[...]
    t = dis * (a[0] + a[1] + hp_ref[...]) + b_ref[...]
    h = _bn_relu(t, g_ref[...], be_ref[...])
    out_ref[...] = dis * lax.dot_general(
        h, W_ref[...], (((1,), (0,)), ((), ())),
        preferred_element_type=jnp.float32)


def _qkv_body(agg_ref, hp_ref, dis_ref, b_ref, g_ref, be_ref, Wqkv_ref,
              bqkv_ref, out_ref):
    a = agg_ref[...]
    t = dis_ref[...] * (a[0] + a[1] + hp_ref[...]) + b_ref[...]
    h = _bn_relu(t, g_ref[...], be_ref[...])
    out_ref[...] = lax.dot_general(
        h, Wqkv_ref[...], (((1,), (1,)), ((), ())),
        preferred_element_type=jnp.float32) + bqkv_ref[...]


_BQ = 200


def _attn_body(qkv_blk, qkv_full, Wo_ref, bo_ref, L1w_ref, L1b_ref,
               L2w_ref, L2b_ref, L3w_ref, L3b_ref, out_ref):
    qb = qkv_blk[...][:, 0:_HID]
    kv = qkv_full[...]
    kf = kv[:, _HID:2 * _HID]
    vf = kv[:, 2 * _HID:3 * _HID]
    scale = 1.0 / np.sqrt(_HID // _NH)
    outs = []
    for hh in range(_NH):
        lo = 8 * hh
        qh = qb[:, lo:lo + 8]
        kh = kf[:, lo:lo + 8]
        vh = vf[:, lo:lo + 8]
        sc = lax.dot_general(qh, kh, (((1,), (1,)), ((), ())),
                             preferred_element_type=jnp.float32) * scale
        m = jnp.max(sc, axis=1, keepdims=True)
        e = jnp.exp(sc - m)
        a = e / jnp.sum(e, axis=1, keepdims=True)
        outs.append(lax.dot_general(a, vh, (((1,), (0,)), ((), ())),
                                    preferred_element_type=jnp.float32))
    o = jnp.concatenate(outs, axis=1)
    h = lax.dot_general(o, Wo_ref[...], (((1,), (1,)), ((), ())),
                        preferred_element_type=jnp.float32) + bo_ref[...]
    h = jnp.maximum(lax.dot_general(h, L1w_ref[...], (((1,), (1,)), ((), ())),
                                    preferred_element_type=jnp.float32)
                    + L1b_ref[...], 0.0)
    h = jnp.maximum(lax.dot_general(h, L2w_ref[...], (((1,), (1,)), ((), ())),
                                    preferred_element_type=jnp.float32)
                    + L2b_ref[...], 0.0)
    out_ref[...] = lax.dot_general(h, L3w_ref[...], (((1,), (1,)), ((), ())),
                                   preferred_element_type=jnp.float32) \
        + L3b_ref[...]


def _full(shape):
    return pl.BlockSpec(shape, lambda i: (0,) * len(shape))


def kernel(x, edge_index, params):
    p = params
    ei = edge_index.astype(jnp.int32)
    x16 = jnp.pad(x, ((0, 0), (0, 16 - x.shape[1])))
    W0p = jnp.pad(p['W0'], ((0, 16 - p['W0'].shape[0]), (0, 0)))
    zeros16 = jnp.zeros((_N, 16), jnp.float32)
    zeros64 = jnp.zeros((_N, _HID), jnp.float32)
    ones_chunk = jnp.ones((_CHUNK, 16), jnp.float32)

    def r2(v):
        return v.reshape(1, -1)

    deg2 = _sc_degree(ei, ones_chunk, zeros16)
    xp, dis = pl.pallas_call(
        _prep_body,
        out_shape=(jax.ShapeDtypeStruct((_N, 16), jnp.float32),
                   jax.ShapeDtypeStruct((_N, 1), jnp.float32)),
    )(deg2, x16)

    aggx = _sc_scatter16(xp, ei, zeros16)
    hp1 = pl.pallas_call(
        _layer0_body,
        out_shape=jax.ShapeDtypeStruct((_N, _HID), jnp.float32),
    )(aggx, xp, dis, W0p, r2(p['b0']), r2(p['g0']), r2(p['be0']), p['W1'])

    agg1 = _sc_scatter64(hp1, ei, zeros64)
    hp2 = pl.pallas_call(
        _layer_body,
        out_shape=jax.ShapeDtypeStruct((_N, _HID), jnp.float32),
    )(agg1, hp1, dis, r2(p['b1']), r2(p['g1']), r2(p['be1']), p['W2'])

    agg2 = _sc_scatter64(hp2, ei, zeros64)
    qkv = pl.pallas_call(
        _qkv_body,
        out_shape=jax.ShapeDtypeStruct((_N, 3 * _HID), jnp.float32),
    )(agg2, hp2, dis, r2(p['b2']), r2(p['g2']), r2(p['be2']), p['Wqkv'],
      r2(p['bqkv']))

    out = pl.pallas_call(
        _attn_body,
        grid=(_N // _BQ,),
        in_specs=[
            pl.BlockSpec((_BQ, 3 * _HID), lambda i: (i, 0)),
            _full((_N, 3 * _HID)),
            _full((_HID, _HID)),
            _full((1, _HID)),
            _full((_HID // 2, _HID)),
            _full((1, _HID // 2)),
            _full((_HID // 4, _HID // 2)),
            _full((1, _HID // 4)),
            _full((1, _HID // 4)),
            _full((1, 1)),
        ],
        out_specs=pl.BlockSpec((_BQ, 1), lambda i: (i, 0)),
        out_shape=jax.ShapeDtypeStruct((_N, 1), jnp.float32),
    )(qkv, qkv, p['Wo'], r2(p['bo']), p['L1w'], r2(p['L1b']), p['L2w'],
      r2(p['L2b']), p['L3w'], r2(p['L3b']))
    return out[:, 0]


# speedup vs baseline: 7.7888x; 7.7888x over previous
import functools

import jax
import jax.numpy as jnp
import numpy as np
from jax import lax
from jax.experimental import pallas as pl
from jax.experimental.pallas import tpu as pltpu
from jax.experimental.pallas import tpu_sc as plsc

_N = 10000
_E = 640000
_HID = 64
_NH = 8
_NC = 2
_NS = 16
_NW = _NC * _NS
_EPT = _E // _NW
_CHUNK = 80
_NCHUNK = _EPT // _CHUNK
_NP = 10112
_RPT = _NP // _NS

_mesh = plsc.VectorSubcoreMesh(core_axis_name="c", subcore_axis_name="s")


@functools.partial(
    pl.kernel,
    mesh=_mesh,
    compiler_params=pltpu.CompilerParams(use_tc_tiling_on_sc=False),
    out_type=jax.ShapeDtypeStruct((_NC, _NP, 16), jnp.float32),
    scratch_types=[
        pltpu.VMEM((_CHUNK,), jnp.int32),
        pltpu.VMEM((_CHUNK, 16), jnp.float32),
        pltpu.VMEM_SHARED((_NP, 16), jnp.float32),
    ],
)
def _sc_degree(dst_hbm, ones_hbm, zeros_hbm, out_hbm, dstv, ones_v, deg_s):
    c = lax.axis_index("c")
    s = lax.axis_index("s")
    row0 = pl.multiple_of(s * _RPT, 8)
    pltpu.sync_copy(zeros_hbm.at[pl.ds(row0, _RPT)],
                    deg_s.at[pl.ds(row0, _RPT)])
    pltpu.sync_copy(ones_hbm, ones_v)
    plsc.subcore_barrier()
    base0 = (c * _NS + s) * _EPT

    def body(j, carry):
        b = pl.multiple_of(base0 + j * _CHUNK, 8)
        pltpu.sync_copy(dst_hbm.at[pl.ds(b, _CHUNK)], dstv)
        pltpu.sync_copy(ones_v, deg_s.at[dstv], add=True)
        return carry

    lax.fori_loop(0, _NCHUNK, body, 0)
    plsc.subcore_barrier()
    pltpu.sync_copy(deg_s.at[pl.ds(row0, _RPT)],
                    out_hbm.at[c, pl.ds(row0, _RPT)])


def _make_edge_scatter(width):
    @functools.partial(
        pl.kernel,
        mesh=_mesh,
        compiler_params=pltpu.CompilerParams(use_tc_tiling_on_sc=False),
        out_type=jax.ShapeDtypeStruct((_NC, _NP, width), jnp.float32),
        scratch_types=[
            pltpu.VMEM((_CHUNK,), jnp.int32),
            pltpu.VMEM((_CHUNK,), jnp.int32),
            pltpu.VMEM((_CHUNK, width), jnp.float32),
            pltpu.VMEM_SHARED((_NP, width), jnp.float32),
            pltpu.SemaphoreType.DMA,
        ],
    )
    def scat(tab_hbm, src_hbm, dst_hbm, zeros_hbm, out_hbm, srcv, dstv, rows, agg_s, sem):
        c = lax.axis_index("c")
        s = lax.axis_index("s")
        row0 = pl.multiple_of(s * _RPT, 8)
        pltpu.sync_copy(zeros_hbm.at[pl.ds(row0, _RPT)],
                        agg_s.at[pl.ds(row0, _RPT)])
        plsc.subcore_barrier()
        base0 = (c * _NS + s) * _EPT

        def body(j, carry):
            b = pl.multiple_of(base0 + j * _CHUNK, 8)
            pltpu.sync_copy(src_hbm.at[pl.ds(b, _CHUNK)], srcv)
            pltpu.sync_copy(dst_hbm.at[pl.ds(b, _CHUNK)], dstv)
            pltpu.async_copy(tab_hbm.at[srcv], rows, sem).wait()
            pltpu.sync_copy(rows, agg_s.at[dstv], add=True)
            return carry

        lax.fori_loop(0, _NCHUNK, body, 0)
        plsc.subcore_barrier()
        pltpu.sync_copy(agg_s.at[pl.ds(row0, _RPT)],
                        out_hbm.at[c, pl.ds(row0, _RPT)])

    return scat


_sc_scatter16 = _make_edge_scatter(16)
_sc_scatter64 = _make_edge_scatter(_HID)


def _prep_body(deg2_ref, x16_ref, xp_ref, dis_ref):
    d = deg2_ref[...]
    deg = d[0, :_N, 0:1] + d[1, :_N, 0:1] + 1.0
    dis = lax.rsqrt(deg)
    dis_ref[...] = dis
    xp_ref[...] = x16_ref[...] * dis


def _bn_relu(t, g, be):
    m = jnp.mean(t, axis=0, keepdims=True)
    v = jnp.mean((t - m) ** 2, axis=0, keepdims=True)
    h = (t - m) * lax.rsqrt(v + 1e-5) * g + be
    return jnp.maximum(h, 0.0)


def _layer0_body(agg_ref, xp_ref, dis_ref, W0_ref, b_ref, g_ref, be_ref,
                 W1_ref, out_ref):
    a = agg_ref[...]
    dis = dis_ref[...]
    t16 = a[0, :_N] + a[1, :_N] + xp_ref[...]
    t = dis * lax.dot_general(t16, W0_ref[...], (((1,), (0,)), ((), ())),
                              preferred_element_type=jnp.float32) + b_ref[...]
    h = _bn_relu(t, g_ref[...], be_ref[...])
    out_ref[...] = dis * lax.dot_general(
        h, W1_ref[...], (((1,), (0,)), ((), ())),
        preferred_element_type=jnp.float32)


def _layer_body(agg_ref, hp_ref, dis_ref, b_ref, g_ref, be_ref, W_ref,
                out_ref):
    a = agg_ref[...]
    dis = dis_ref[...]
    t = dis * (a[0, :_N] + a[1, :_N] + hp_ref[...]) + b_ref[...]
    h = _bn_relu(t, g_ref[...], be_ref[...])
    out_ref[...] = dis * lax.dot_general(
        h, W_ref[...], (((1,), (0,)), ((), ())),
        preferred_element_type=jnp.float32)


def _qkv_body(agg_ref, hp_ref, dis_ref, b_ref, g_ref, be_ref, Wqkv_ref,
              bqkv_ref, out_ref):
    a = agg_ref[...]
    t = dis_ref[...] * (a[0, :_N] + a[1, :_N] + hp_ref[...]) + b_ref[...]
    h = _bn_relu(t, g_ref[...], be_ref[...])
    out_ref[...] = lax.dot_general(
        h, Wqkv_ref[...], (((1,), (1,)), ((), ())),
        preferred_element_type=jnp.float32) + bqkv_ref[...]


_BQ = 200


def _attn_body(qkv_blk, qkv_full, Wo_ref, bo_ref, L1w_ref, L1b_ref,
               L2w_ref, L2b_ref, L3w_ref, L3b_ref, out_ref):
    qb = qkv_blk[...][:, 0:_HID]
    kv = qkv_full[...]
    kf = kv[:, _HID:2 * _HID]
    vf = kv[:, 2 * _HID:3 * _HID]
    scale = 1.0 / np.sqrt(_HID // _NH)
    outs = []
    for hh in range(_NH):
        lo = 8 * hh
        qh = qb[:, lo:lo + 8]
        kh = kf[:, lo:lo + 8]
        vh = vf[:, lo:lo + 8]
        sc = lax.dot_general(qh, kh, (((1,), (1,)), ((), ())),
                             preferred_element_type=jnp.float32) * scale
        m = jnp.max(sc, axis=1, keepdims=True)
        e = jnp.exp(sc - m)
        a = e / jnp.sum(e, axis=1, keepdims=True)
        outs.append(lax.dot_general(a, vh, (((1,), (0,)), ((), ())),
                                    preferred_element_type=jnp.float32))
    o = jnp.concatenate(outs, axis=1)
    h = lax.dot_general(o, Wo_ref[...], (((1,), (1,)), ((), ())),
                        preferred_element_type=jnp.float32) + bo_ref[...]
    h = jnp.maximum(lax.dot_general(h, L1w_ref[...], (((1,), (1,)), ((), ())),
                                    preferred_element_type=jnp.float32)
                    + L1b_ref[...], 0.0)
    h = jnp.maximum(lax.dot_general(h, L2w_ref[...], (((1,), (1,)), ((), ())),
                                    preferred_element_type=jnp.float32)
                    + L2b_ref[...], 0.0)
    out_ref[...] = jnp.sum(h * L3w_ref[...], axis=1, keepdims=True) \
        + L3b_ref[0, 0]


def _full(shape):
    return pl.BlockSpec(shape, lambda i: (0,) * len(shape))


def kernel(x, edge_index, params):
    p = params
    ei = edge_index.astype(jnp.int32)
    e_src, e_dst = ei[0], ei[1]
    x16 = jnp.pad(x, ((0, 0), (0, 16 - x.shape[1])))
    W0p = jnp.pad(p['W0'], ((0, 16 - p['W0'].shape[0]), (0, 0)))
    zeros16 = jnp.zeros((_NP, 16), jnp.float32)
    zeros64 = jnp.zeros((_NP, _HID), jnp.float32)
    ones_chunk = jnp.ones((_CHUNK, 16), jnp.float32)

    def r2(v):
        return v.reshape(1, -1)

    deg2 = _sc_degree(e_dst, ones_chunk, zeros16)
    xp, dis = pl.pallas_call(
        _prep_body,
        out_shape=(jax.ShapeDtypeStruct((_N, 16), jnp.float32),
                   jax.ShapeDtypeStruct((_N, 1), jnp.float32)),
    )(deg2, x16)

    aggx = _sc_scatter16(xp, e_src, e_dst, zeros16)
    hp1 = pl.pallas_call(
        _layer0_body,
        out_shape=jax.ShapeDtypeStruct((_N, _HID), jnp.float32),
    )(aggx, xp, dis, W0p, r2(p['b0']), r2(p['g0']), r2(p['be0']), p['W1'])

    agg1 = _sc_scatter64(hp1, e_src, e_dst, zeros64)
    hp2 = pl.pallas_call(
        _layer_body,
        out_shape=jax.ShapeDtypeStruct((_N, _HID), jnp.float32),
    )(agg1, hp1, dis, r2(p['b1']), r2(p['g1']), r2(p['be1']), p['W2'])

    agg2 = _sc_scatter64(hp2, e_src, e_dst, zeros64)
    qkv = pl.pallas_call(
        _qkv_body,
        out_shape=jax.ShapeDtypeStruct((_N, 3 * _HID), jnp.float32),
    )(agg2, hp2, dis, r2(p['b2']), r2(p['g2']), r2(p['be2']), p['Wqkv'],
      r2(p['bqkv']))

    out = pl.pallas_call(
        _attn_body,
        grid=(_N // _BQ,),
        in_specs=[
            pl.BlockSpec((_BQ, 3 * _HID), lambda i: (i, 0)),
            _full((_N, 3 * _HID)),
            _full((_HID, _HID)),
            _full((1, _HID)),
            _full((_HID // 2, _HID)),
            _full((1, _HID // 2)),
            _full((_HID // 4, _HID // 2)),
            _full((1, _HID // 4)),
            _full((1, _HID // 4)),
            _full((1, 1)),
        ],
        out_specs=pl.BlockSpec((_BQ, 1), lambda i: (i, 0)),
        out_shape=jax.ShapeDtypeStruct((_N, 1), jnp.float32),
    )(qkv, qkv, p['Wo'], r2(p['bo']), p['L1w'], r2(p['L1b']), p['L2w'],
      r2(p['L2b']), p['L3w'], r2(p['L3b']))
    return out[:, 0]


# speedup vs baseline: 11.6786x; 1.4994x over previous
import functools

import jax
import jax.numpy as jnp
import numpy as np
from jax import lax
from jax.experimental import pallas as pl
from jax.experimental.pallas import tpu as pltpu
from jax.experimental.pallas import tpu_sc as plsc

_N = 10000
_E = 640000
_HID = 64
_NH = 8
_NC = 2
_NS = 16
_NW = _NC * _NS
_EPT = _E // _NW
_CHUNK = 128
_NCHUNK = -(-_EPT // _CHUNK)
_EPTP = _NCHUNK * _CHUNK
_NP = 10112
_RPT = _NP // _NS
_DUMP = _N + 64
_NBUF = 5

_mesh = plsc.VectorSubcoreMesh(core_axis_name="c", subcore_axis_name="s")


@functools.partial(
    pl.kernel,
    mesh=_mesh,
    compiler_params=pltpu.CompilerParams(use_tc_tiling_on_sc=False),
    out_type=jax.ShapeDtypeStruct((_NC, _NP, 16), jnp.float32),
    scratch_types=[
        pltpu.VMEM((_NCHUNK, _CHUNK), jnp.int32),
        pltpu.VMEM((_CHUNK, 16), jnp.float32),
        pltpu.VMEM_SHARED((_NP, 16), jnp.float32),
    ],
)
def _sc_degree(dst3_hbm, ones_hbm, zeros_hbm, out_hbm, dst_all, ones_v, deg_s):
    c = lax.axis_index("c")
    s = lax.axis_index("s")
    wid = c * _NS + s
    row0 = pl.multiple_of(s * _RPT, 8)
    pltpu.sync_copy(zeros_hbm.at[pl.ds(row0, _RPT)],
                    deg_s.at[pl.ds(row0, _RPT)])
    pltpu.sync_copy(ones_hbm, ones_v)
    pltpu.sync_copy(dst3_hbm.at[wid], dst_all)
    plsc.subcore_barrier()

    def body(j, carry):
        pltpu.sync_copy(ones_v, deg_s.at[dst_all.at[j]], add=True)
        return carry

    lax.fori_loop(0, _NCHUNK, body, 0)
    plsc.subcore_barrier()
    pltpu.sync_copy(deg_s.at[pl.ds(row0, _RPT)],
                    out_hbm.at[c, pl.ds(row0, _RPT)])


def _make_edge_scatter(width):
    @functools.partial(
        pl.kernel,
        mesh=_mesh,
        compiler_params=pltpu.CompilerParams(use_tc_tiling_on_sc=False),
        out_type=jax.ShapeDtypeStruct((_NC, _NP, width), jnp.float32),
        scratch_types=[
            pltpu.VMEM((_NCHUNK, _CHUNK), jnp.int32),
            pltpu.VMEM((_NCHUNK, _CHUNK), jnp.int32),
            pltpu.VMEM((_NBUF, _CHUNK, width), jnp.float32),
            pltpu.VMEM_SHARED((_NP, width), jnp.float32),
            [pltpu.SemaphoreType.DMA] * _NBUF,
        ],
    )
    def scat(tab_hbm, src3_hbm, dst3_hbm, zeros_hbm, out_hbm, src_all,
             dst_all, rows, agg_s, gsems):
        c = lax.axis_index("c")
        s = lax.axis_index("s")
        wid = c * _NS + s
        row0 = pl.multiple_of(s * _RPT, 8)
        pltpu.sync_copy(zeros_hbm.at[pl.ds(row0, _RPT)],
                        agg_s.at[pl.ds(row0, _RPT)])
        pltpu.sync_copy(src3_hbm.at[wid], src_all)
        pltpu.sync_copy(dst3_hbm.at[wid], dst_all)
        plsc.subcore_barrier()

        for b in range(_NBUF):
            pltpu.async_copy(tab_hbm.at[src_all.at[b]], rows.at[b], gsems[b])

        def body(j, carry):
            for b in range(_NBUF):

                @pl.when(j % _NBUF == b)
                def _():
                    pltpu.make_async_copy(
                        tab_hbm.at[pl.ds(0, _CHUNK)], rows.at[b],
                        gsems[b]).wait()
                    pltpu.sync_copy(rows.at[b], agg_s.at[dst_all.at[j]],
                                    add=True)

                    @pl.when(j + _NBUF < _NCHUNK)
                    def _():
                        pltpu.async_copy(tab_hbm.at[src_all.at[j + _NBUF]],
                                         rows.at[b], gsems[b])

            return carry

        lax.fori_loop(0, _NCHUNK, body, 0)
        plsc.subcore_barrier()
        pltpu.sync_copy(agg_s.at[pl.ds(row0, _RPT)],
                        out_hbm.at[c, pl.ds(row0, _RPT)])

    return scat


_sc_scatter16 = _make_edge_scatter(16)
_sc_scatter64 = _make_edge_scatter(_HID)


def _prep_body(deg2_ref, x16_ref, xp_ref, dis_ref):
    d = deg2_ref[...]
    deg = d[0, :_N, 0:1] + d[1, :_N, 0:1] + 1.0
    dis = lax.rsqrt(deg)
    dis_ref[...] = dis
    xp_ref[...] = x16_ref[...] * dis


def _bn_relu(t, g, be):
    m = jnp.mean(t, axis=0, keepdims=True)
    v = jnp.mean((t - m) ** 2, axis=0, keepdims=True)
    h = (t - m) * lax.rsqrt(v + 1e-5) * g + be
    return jnp.maximum(h, 0.0)


def _layer0_body(agg_ref, xp_ref, dis_ref, W0_ref, b_ref, g_ref, be_ref,
                 W1_ref, out_ref):
    a = agg_ref[...]
    dis = dis_ref[...]
    t16 = a[0, :_N] + a[1, :_N] + xp_ref[...]
    t = dis * lax.dot_general(t16, W0_ref[...], (((1,), (0,)), ((), ())),
                              preferred_element_type=jnp.float32) + b_ref[...]
    h = _bn_relu(t, g_ref[...], be_ref[...])
    out_ref[...] = dis * lax.dot_general(
        h, W1_ref[...], (((1,), (0,)), ((), ())),
        preferred_element_type=jnp.float32)


def _layer_body(agg_ref, hp_ref, dis_ref, b_ref, g_ref, be_ref, W_ref,
                out_ref):
    a = agg_ref[...]
    dis = dis_ref[...]
    t = dis * (a[0, :_N] + a[1, :_N] + hp_ref[...]) + b_ref[...]
    h = _bn_relu(t, g_ref[...], be_ref[...])
    out_ref[...] = dis * lax.dot_general(
        h, W_ref[...], (((1,), (0,)), ((), ())),
        preferred_element_type=jnp.float32)


def _qkv_body(agg_ref, hp_ref, dis_ref, b_ref, g_ref, be_ref, Wqkv_ref,
              bqkv_ref, out_ref):
    a = agg_ref[...]
    t = dis_ref[...] * (a[0, :_N] + a[1, :_N] + hp_ref[...]) + b_ref[...]
    h = _bn_relu(t, g_ref[...], be_ref[...])
    out_ref[...] = lax.dot_general(
        h, Wqkv_ref[...], (((1,), (1,)), ((), ())),
        preferred_element_type=jnp.float32) + bqkv_ref[...]


_BQ = 200


def _attn_body(qkv_blk, qkv_full, Wo_ref, bo_ref, L1w_ref, L1b_ref,
               L2w_ref, L2b_ref, L3w_ref, L3b_ref, out_ref):
    qb = qkv_blk[...][:, 0:_HID]
    kv = qkv_full[...]
    kf = kv[:, _HID:2 * _HID]
    vf = kv[:, 2 * _HID:3 * _HID]
    scale = 1.0 / np.sqrt(_HID // _NH)
    outs = []
    for hh in range(_NH):
        lo = 8 * hh
        qh = qb[:, lo:lo + 8]
        kh = kf[:, lo:lo + 8]
        vh = vf[:, lo:lo + 8]
        sc = lax.dot_general(qh, kh, (((1,), (1,)), ((), ())),
                             preferred_element_type=jnp.float32) * scale
        m = jnp.max(sc, axis=1, keepdims=True)
        e = jnp.exp(sc - m)
        a = e / jnp.sum(e, axis=1, keepdims=True)
        outs.append(lax.dot_general(a, vh, (((1,), (0,)), ((), ())),
                                    preferred_element_type=jnp.float32))
    o = jnp.concatenate(outs, axis=1)
    h = lax.dot_general(o, Wo_ref[...], (((1,), (1,)), ((), ())),
                        preferred_element_type=jnp.float32) + bo_ref[...]
    h = jnp.maximum(lax.dot_general(h, L1w_ref[...], (((1,), (1,)), ((), ())),
                                    preferred_element_type=jnp.float32)
                    + L1b_ref[...], 0.0)
    h = jnp.maximum(lax.dot_general(h, L2w_ref[...], (((1,), (1,)), ((), ())),
                                    preferred_element_type=jnp.float32)
                    + L2b_ref[...], 0.0)
    out_ref[...] = jnp.sum(h * L3w_ref[...], axis=1, keepdims=True) \
        + L3b_ref[0, 0]


def _full(shape):
    return pl.BlockSpec(shape, lambda i: (0,) * len(shape))


def kernel(x, edge_index, params):
    p = params
    ei = edge_index.astype(jnp.int32)
    pad = _NW * _EPTP - _E
    e_src = jnp.concatenate(
        [ei[0], jnp.zeros((pad,), jnp.int32)]).reshape(_NW, _NCHUNK, _CHUNK)
    e_dst = jnp.concatenate(
        [ei[1], jnp.full((pad,), _DUMP, jnp.int32)]).reshape(
            _NW, _NCHUNK, _CHUNK)
    x16 = jnp.pad(x, ((0, 0), (0, 16 - x.shape[1])))
    W0p = jnp.pad(p['W0'], ((0, 16 - p['W0'].shape[0]), (0, 0)))
    zeros16 = jnp.zeros((_NP, 16), jnp.float32)
    zeros64 = jnp.zeros((_NP, _HID), jnp.float32)
    ones_chunk = jnp.ones((_CHUNK, 16), jnp.float32)

    def r2(v):
        return v.reshape(1, -1)

    deg2 = _sc_degree(e_dst, ones_chunk, zeros16)
    xp, dis = pl.pallas_call(
        _prep_body,
        out_shape=(jax.ShapeDtypeStruct((_N, 16), jnp.float32),
                   jax.ShapeDtypeStruct((_N, 1), jnp.float32)),
    )(deg2, x16)

    aggx = _sc_scatter16(xp, e_src, e_dst, zeros16)
    hp1 = pl.pallas_call(
        _layer0_body,
        out_shape=jax.ShapeDtypeStruct((_N, _HID), jnp.float32),
    )(aggx, xp, dis, W0p, r2(p['b0']), r2(p['g0']), r2(p['be0']), p['W1'])

    agg1 = _sc_scatter64(hp1, e_src, e_dst, zeros64)
    hp2 = pl.pallas_call(
        _layer_body,
        out_shape=jax.ShapeDtypeStruct((_N, _HID), jnp.float32),
    )(agg1, hp1, dis, r2(p['b1']), r2(p['g1']), r2(p['be1']), p['W2'])

    agg2 = _sc_scatter64(hp2, e_src, e_dst, zeros64)
    qkv = pl.pallas_call(
        _qkv_body,
        out_shape=jax.ShapeDtypeStruct((_N, 3 * _HID), jnp.float32),
    )(agg2, hp2, dis, r2(p['b2']), r2(p['g2']), r2(p['be2']), p['Wqkv'],
      r2(p['bqkv']))

    out = pl.pallas_call(
        _attn_body,
        grid=(_N // _BQ,),
        in_specs=[
            pl.BlockSpec((_BQ, 3 * _HID), lambda i: (i, 0)),
            _full((_N, 3 * _HID)),
            _full((_HID, _HID)),
            _full((1, _HID)),
            _full((_HID // 2, _HID)),
            _full((1, _HID // 2)),
            _full((_HID // 4, _HID // 2)),
            _full((1, _HID // 4)),
            _full((1, _HID // 4)),
            _full((1, 1)),
        ],
        out_specs=pl.BlockSpec((_BQ, 1), lambda i: (i, 0)),
        out_shape=jax.ShapeDtypeStruct((_N, 1), jnp.float32),
    )(qkv, qkv, p['Wo'], r2(p['bo']), p['L1w'], r2(p['L1b']), p['L2w'],
      r2(p['L2b']), p['L3w'], r2(p['L3b']))
    return out[:, 0]


# speedup vs baseline: 13.7778x; 1.1797x over previous
import functools

import jax
import jax.numpy as jnp
import numpy as np
from jax import lax
from jax.experimental import pallas as pl
from jax.experimental.pallas import tpu as pltpu
from jax.experimental.pallas import tpu_sc as plsc

_N = 10000
_E = 640000
_HID = 64
_NH = 8
_NC = 2
_NS = 16
_NW = _NC * _NS
_EPT = _E // _NW
_CHUNK = 128
_NCHUNK = -(-_EPT // _CHUNK)
_EPTP = _NCHUNK * _CHUNK
_NP = 10112
_RPT = _NP // _NS
_DUMP = _N + 64
_NBUF = 5

_mesh = plsc.VectorSubcoreMesh(core_axis_name="c", subcore_axis_name="s")


@functools.partial(
    pl.kernel,
    mesh=_mesh,
    compiler_params=pltpu.CompilerParams(use_tc_tiling_on_sc=False),
    out_type=jax.ShapeDtypeStruct((_NC, _NP, 16), jnp.float32),
    scratch_types=[
        pltpu.VMEM((_NCHUNK, _CHUNK), jnp.int32),
        pltpu.VMEM((_CHUNK, 16), jnp.float32),
        pltpu.VMEM_SHARED((_NP, 16), jnp.float32),
    ],
)
def _sc_degree(dst3_hbm, ones_hbm, zeros_hbm, out_hbm, dst_all, ones_v, deg_s):
    c = lax.axis_index("c")
    s = lax.axis_index("s")
    wid = c * _NS + s
    row0 = pl.multiple_of(s * _RPT, 8)
    pltpu.sync_copy(zeros_hbm.at[pl.ds(row0, _RPT)],
                    deg_s.at[pl.ds(row0, _RPT)])
    pltpu.sync_copy(ones_hbm, ones_v)
    pltpu.sync_copy(dst3_hbm.at[wid], dst_all)
    plsc.subcore_barrier()

    def body(j, carry):
        pltpu.sync_copy(ones_v, deg_s.at[dst_all.at[j]], add=True)
        return carry

    lax.fori_loop(0, _NCHUNK, body, 0)
    plsc.subcore_barrier()
    pltpu.sync_copy(deg_s.at[pl.ds(row0, _RPT)],
                    out_hbm.at[c, pl.ds(row0, _RPT)])


def _make_edge_scatter(width):
    @functools.partial(
        pl.kernel,
        mesh=_mesh,
        compiler_params=pltpu.CompilerParams(use_tc_tiling_on_sc=False),
        out_type=jax.ShapeDtypeStruct((_NC, _NP, width), jnp.float32),
        scratch_types=[
            pltpu.VMEM((_NCHUNK, _CHUNK), jnp.int32),
            pltpu.VMEM((_NCHUNK, _CHUNK), jnp.int32),
            pltpu.VMEM((_NBUF, _CHUNK, width), jnp.float32),
            pltpu.VMEM_SHARED((_NP, width), jnp.float32),
            [pltpu.SemaphoreType.DMA] * _NBUF,
        ],
    )
    def scat(tab_hbm, src3_hbm, dst3_hbm, zeros_hbm, out_hbm, src_all,
             dst_all, rows, agg_s, gsems):
        c = lax.axis_index("c")
        s = lax.axis_index("s")
        wid = c * _NS + s
        row0 = pl.multiple_of(s * _RPT, 8)
        pltpu.sync_copy(zeros_hbm.at[pl.ds(row0, _RPT)],
                        agg_s.at[pl.ds(row0, _RPT)])
        pltpu.sync_copy(src3_hbm.at[wid], src_all)
        pltpu.sync_copy(dst3_hbm.at[wid], dst_all)
        plsc.subcore_barrier()

        for b in range(_NBUF):
            pltpu.async_copy(tab_hbm.at[src_all.at[b]], rows.at[b], gsems[b])

        def body(j, carry):
            for b in range(_NBUF):

                @pl.when(j % _NBUF == b)
                def _():
                    pltpu.make_async_copy(
                        tab_hbm.at[pl.ds(0, _CHUNK)], rows.at[b],
                        gsems[b]).wait()
                    pltpu.sync_copy(rows.at[b], agg_s.at[dst_all.at[j]],
                                    add=True)

                    @pl.when(j + _NBUF < _NCHUNK)
                    def _():
                        pltpu.async_copy(tab_hbm.at[src_all.at[j + _NBUF]],
                                         rows.at[b], gsems[b])

            return carry

        lax.fori_loop(0, _NCHUNK, body, 0)
        plsc.subcore_barrier()
        pltpu.sync_copy(agg_s.at[pl.ds(row0, _RPT)],
                        out_hbm.at[c, pl.ds(row0, _RPT)])

    return scat


_sc_scatter16 = _make_edge_scatter(16)
_sc_scatter64 = _make_edge_scatter(_HID)


def _prep_body(deg2_ref, x16_ref, xp_ref, dis_ref):
    d = deg2_ref[...]
    deg = d[0, :_N, 0:1] + d[1, :_N, 0:1] + 1.0
    dis = lax.rsqrt(deg)
    dis_ref[...] = dis
    xp_ref[...] = x16_ref[...] * dis


def _bn_relu(t, g, be):
    m = jnp.mean(t, axis=0, keepdims=True)
    v = jnp.mean((t - m) ** 2, axis=0, keepdims=True)
    h = (t - m) * lax.rsqrt(v + 1e-5) * g + be
    return jnp.maximum(h, 0.0)


def _layer0_body(agg_ref, xp_ref, dis_ref, W0_ref, b_ref, g_ref, be_ref,
                 W1_ref, out_ref):
    a = agg_ref[...]
    dis = dis_ref[...]
    t16 = a[0, :_N] + a[1, :_N] + xp_ref[...]
    t = dis * lax.dot_general(t16, W0_ref[...], (((1,), (0,)), ((), ())),
                              preferred_element_type=jnp.float32) + b_ref[...]
    h = _bn_relu(t, g_ref[...], be_ref[...])
    out_ref[...] = dis * lax.dot_general(
        h, W1_ref[...], (((1,), (0,)), ((), ())),
        preferred_element_type=jnp.float32)


def _layer_body(agg_ref, hp_ref, dis_ref, b_ref, g_ref, be_ref, W_ref,
                out_ref):
    a = agg_ref[...]
    dis = dis_ref[...]
    t = dis * (a[0, :_N] + a[1, :_N] + hp_ref[...]) + b_ref[...]
    h = _bn_relu(t, g_ref[...], be_ref[...])
    out_ref[...] = dis * lax.dot_general(
        h, W_ref[...], (((1,), (0,)), ((), ())),
        preferred_element_type=jnp.float32)


def _qkv_body(agg_ref, hp_ref, dis_ref, b_ref, g_ref, be_ref, Wqkv_ref,
              bqkv_ref, out_ref):
    a = agg_ref[...]
    t = dis_ref[...] * (a[0, :_N] + a[1, :_N] + hp_ref[...]) + b_ref[...]
    h = _bn_relu(t, g_ref[...], be_ref[...])
    out_ref[...] = lax.dot_general(
        h, Wqkv_ref[...], (((1,), (1,)), ((), ())),
        preferred_element_type=jnp.float32) + bqkv_ref[...]


_BQ = 256
_NQP = 10240


def _attn_body(qkv_blk, qkv_full, Wo_ref, bo_ref, L1w_ref, L1b_ref,
               L2w_ref, L2b_ref, L3w_ref, L3b_ref, out_ref):
    qb = qkv_blk[...][:, 0:_HID]
    kv = qkv_full[...]
    kf = kv[:, _HID:2 * _HID]
    vf = kv[:, 2 * _HID:3 * _HID]
    scale = 1.0 / np.sqrt(_HID // _NH)
    outs = []
    for hh in range(_NH):
        lo = 8 * hh
        qh = qb[:, lo:lo + 8]
        kh = kf[:, lo:lo + 8]
        vh = vf[:, lo:lo + 8]
        sc = lax.dot_general(qh, kh, (((1,), (1,)), ((), ())),
                             preferred_element_type=jnp.float32) * scale
        m = jnp.max(sc, axis=1, keepdims=True)
        e = jnp.exp(sc - m)
        s = jnp.sum(e, axis=1, keepdims=True)
        outs.append(lax.dot_general(e, vh, (((1,), (0,)), ((), ())),
                                    preferred_element_type=jnp.float32) / s)
    o = jnp.concatenate(outs, axis=1)
    h = lax.dot_general(o, Wo_ref[...], (((1,), (1,)), ((), ())),
                        preferred_element_type=jnp.float32) + bo_ref[...]
    h = jnp.maximum(lax.dot_general(h, L1w_ref[...], (((1,), (1,)), ((), ())),
                                    preferred_element_type=jnp.float32)
                    + L1b_ref[...], 0.0)
    h = jnp.maximum(lax.dot_general(h, L2w_ref[...], (((1,), (1,)), ((), ())),
                                    preferred_element_type=jnp.float32)
                    + L2b_ref[...], 0.0)
    out_ref[...] = jnp.sum(h * L3w_ref[...], axis=1, keepdims=True) \
        + L3b_ref[0, 0]


def _full(shape):
    return pl.BlockSpec(shape, lambda i: (0,) * len(shape))


def kernel(x, edge_index, params):
    p = params
    ei = edge_index.astype(jnp.int32)
    pad = _NW * _EPTP - _E
    e_src = jnp.concatenate(
        [ei[0], jnp.zeros((pad,), jnp.int32)]).reshape(_NW, _NCHUNK, _CHUNK)
    e_dst = jnp.concatenate(
        [ei[1], jnp.full((pad,), _DUMP, jnp.int32)]).reshape(
            _NW, _NCHUNK, _CHUNK)
    x16 = jnp.pad(x, ((0, 0), (0, 16 - x.shape[1])))
    W0p = jnp.pad(p['W0'], ((0, 16 - p['W0'].shape[0]), (0, 0)))
    zeros16 = jnp.zeros((_NP, 16), jnp.float32)
    zeros64 = jnp.zeros((_NP, _HID), jnp.float32)
    ones_chunk = jnp.ones((_CHUNK, 16), jnp.float32)

    def r2(v):
        return v.reshape(1, -1)

    deg2 = _sc_degree(e_dst, ones_chunk, zeros16)
    xp, dis = pl.pallas_call(
        _prep_body,
        out_shape=(jax.ShapeDtypeStruct((_N, 16), jnp.float32),
                   jax.ShapeDtypeStruct((_N, 1), jnp.float32)),
    )(deg2, x16)

    aggx = _sc_scatter16(xp, e_src, e_dst, zeros16)
    hp1 = pl.pallas_call(
        _layer0_body,
        out_shape=jax.ShapeDtypeStruct((_N, _HID), jnp.float32),
    )(aggx, xp, dis, W0p, r2(p['b0']), r2(p['g0']), r2(p['be0']), p['W1'])

    agg1 = _sc_scatter64(hp1, e_src, e_dst, zeros64)
    hp2 = pl.pallas_call(
        _layer_body,
        out_shape=jax.ShapeDtypeStruct((_N, _HID), jnp.float32),
    )(agg1, hp1, dis, r2(p['b1']), r2(p['g1']), r2(p['be1']), p['W2'])

    agg2 = _sc_scatter64(hp2, e_src, e_dst, zeros64)
    qkv = pl.pallas_call(
        _qkv_body,
        out_shape=jax.ShapeDtypeStruct((_N, 3 * _HID), jnp.float32),
    )(agg2, hp2, dis, r2(p['b2']), r2(p['g2']), r2(p['be2']), p['Wqkv'],
      r2(p['bqkv']))

    qkvp = jnp.pad(qkv, ((0, _NQP - _N), (0, 0)))
    out = pl.pallas_call(
        _attn_body,
        grid=(_NQP // _BQ,),
        in_specs=[
            pl.BlockSpec((_BQ, 3 * _HID), lambda i: (i, 0)),
            _full((_N, 3 * _HID)),
            _full((_HID, _HID)),
            _full((1, _HID)),
            _full((_HID // 2, _HID)),
            _full((1, _HID // 2)),
            _full((_HID // 4, _HID // 2)),
            _full((1, _HID // 4)),
            _full((1, _HID // 4)),
            _full((1, 1)),
        ],
        out_specs=pl.BlockSpec((_BQ, 1), lambda i: (i, 0)),
        out_shape=jax.ShapeDtypeStruct((_NQP, 1), jnp.float32),
    )(qkvp, qkv, p['Wo'], r2(p['bo']), p['L1w'], r2(p['L1b']), p['L2w'],
      r2(p['L2b']), p['L3w'], r2(p['L3b']))
    return out[:_N, 0]


# speedup vs baseline: 15.2538x; 1.1071x over previous
import functools

import jax
import jax.numpy as jnp
import numpy as np
from jax import lax
from jax.experimental import pallas as pl
from jax.experimental.pallas import tpu as pltpu
from jax.experimental.pallas import tpu_sc as plsc

_N = 10000
_E = 640000
_HID = 64
_NH = 8
_NC = 2
_NS = 16
_NW = _NC * _NS
_EPT = _E // _NW
_CHUNK = 128
_NCHUNK = -(-_EPT // _CHUNK)
_EPTP = _NCHUNK * _CHUNK
_NP = 10112
_RPT = _NP // _NS
_DUMP = _N + 64
_NBUF = 5

_mesh = plsc.VectorSubcoreMesh(core_axis_name="c", subcore_axis_name="s")


@functools.partial(
    pl.kernel,
    mesh=_mesh,
    compiler_params=pltpu.CompilerParams(use_tc_tiling_on_sc=False),
    out_type=jax.ShapeDtypeStruct((_NC, _NP, 16), jnp.float32),
    scratch_types=[
        pltpu.VMEM((_NCHUNK, _CHUNK), jnp.int32),
        pltpu.VMEM((_CHUNK, 16), jnp.float32),
        pltpu.VMEM_SHARED((_NP, 16), jnp.float32),
    ],
)
def _sc_degree(dst3_hbm, ones_hbm, zeros_hbm, out_hbm, dst_all, ones_v, deg_s):
    c = lax.axis_index("c")
    s = lax.axis_index("s")
    wid = c * _NS + s
    row0 = pl.multiple_of(s * _RPT, 8)
    pltpu.sync_copy(zeros_hbm.at[pl.ds(row0, _RPT)],
                    deg_s.at[pl.ds(row0, _RPT)])
    pltpu.sync_copy(ones_hbm, ones_v)
    pltpu.sync_copy(dst3_hbm.at[wid], dst_all)
    plsc.subcore_barrier()

    def body(j, carry):
        pltpu.sync_copy(ones_v, deg_s.at[dst_all.at[j]], add=True)
        return carry

    lax.fori_loop(0, _NCHUNK, body, 0)
    plsc.subcore_barrier()
    pltpu.sync_copy(deg_s.at[pl.ds(row0, _RPT)],
                    out_hbm.at[c, pl.ds(row0, _RPT)])


def _make_edge_scatter(width):
    @functools.partial(
        pl.kernel,
        mesh=_mesh,
        compiler_params=pltpu.CompilerParams(use_tc_tiling_on_sc=False),
        out_type=jax.ShapeDtypeStruct((_NC, _NP, width), jnp.float32),
        scratch_types=[
            pltpu.VMEM((_NCHUNK, _CHUNK), jnp.int32),
            pltpu.VMEM((_NCHUNK, _CHUNK), jnp.int32),
            pltpu.VMEM((_NBUF, _CHUNK, width), jnp.float32),
            pltpu.VMEM_SHARED((_NP, width), jnp.float32),
            [pltpu.SemaphoreType.DMA] * _NBUF,
        ],
    )
    def scat(tab_hbm, src3_hbm, dst3_hbm, zeros_hbm, out_hbm, src_all,
             dst_all, rows, agg_s, gsems):
        c = lax.axis_index("c")
        s = lax.axis_index("s")
        wid = c * _NS + s
        row0 = pl.multiple_of(s * _RPT, 8)
        pltpu.sync_copy(zeros_hbm.at[pl.ds(row0, _RPT)],
                        agg_s.at[pl.ds(row0, _RPT)])
        pltpu.sync_copy(src3_hbm.at[wid], src_all)
        pltpu.sync_copy(dst3_hbm.at[wid], dst_all)
        plsc.subcore_barrier()

        for b in range(_NBUF):
            pltpu.async_copy(tab_hbm.at[src_all.at[b]], rows.at[b], gsems[b])

        def body(j, carry):
            for b in range(_NBUF):

                @pl.when(j % _NBUF == b)
                def _():
                    pltpu.make_async_copy(
                        tab_hbm.at[pl.ds(0, _CHUNK)], rows.at[b],
                        gsems[b]).wait()
                    pltpu.sync_copy(rows.at[b], agg_s.at[dst_all.at[j]],
                                    add=True)

                    @pl.when(j + _NBUF < _NCHUNK)
                    def _():
                        pltpu.async_copy(tab_hbm.at[src_all.at[j + _NBUF]],
                                         rows.at[b], gsems[b])

            return carry

        lax.fori_loop(0, _NCHUNK, body, 0)
        plsc.subcore_barrier()
        pltpu.sync_copy(agg_s.at[pl.ds(row0, _RPT)],
                        out_hbm.at[c, pl.ds(row0, _RPT)])

    return scat


_sc_scatter16 = _make_edge_scatter(16)
_sc_scatter64 = _make_edge_scatter(_HID)


def _prep_body(deg2_ref, x16_ref, xp_ref, dis_ref):
    d = deg2_ref[...]
    deg = d[0, :_N, 0:1] + d[1, :_N, 0:1] + 1.0
    dis = lax.rsqrt(deg)
    dis_ref[...] = dis
    xp_ref[...] = x16_ref[...] * dis


def _bn_relu(t, g, be):
    m = jnp.mean(t, axis=0, keepdims=True)
    v = jnp.mean((t - m) ** 2, axis=0, keepdims=True)
    h = (t - m) * lax.rsqrt(v + 1e-5) * g + be
    return jnp.maximum(h, 0.0)


def _layer0_body(agg_ref, xp_ref, dis_ref, W0_ref, b_ref, g_ref, be_ref,
                 W1_ref, out_ref):
    a = agg_ref[...]
    dis = dis_ref[...]
    t16 = a[0, :_N] + a[1, :_N] + xp_ref[...]
    t = dis * lax.dot_general(t16, W0_ref[...], (((1,), (0,)), ((), ())),
                              preferred_element_type=jnp.float32) + b_ref[...]
    h = _bn_relu(t, g_ref[...], be_ref[...])
    out_ref[...] = dis * lax.dot_general(
        h, W1_ref[...], (((1,), (0,)), ((), ())),
        preferred_element_type=jnp.float32)


def _layer_body(agg_ref, hp_ref, dis_ref, b_ref, g_ref, be_ref, W_ref,
                out_ref):
    a = agg_ref[...]
    dis = dis_ref[...]
    t = dis * (a[0, :_N] + a[1, :_N] + hp_ref[...]) + b_ref[...]
    h = _bn_relu(t, g_ref[...], be_ref[...])
    out_ref[...] = dis * lax.dot_general(
        h, W_ref[...], (((1,), (0,)), ((), ())),
        preferred_element_type=jnp.float32)


def _qkv_body(agg_ref, hp_ref, dis_ref, b_ref, g_ref, be_ref, Wqkv_ref,
              bqkv_ref, out_ref):
    a = agg_ref[...]
    t = dis_ref[...] * (a[0, :_N] + a[1, :_N] + hp_ref[...]) + b_ref[...]
    h = _bn_relu(t, g_ref[...], be_ref[...])
    out_ref[...] = lax.dot_general(
        h, Wqkv_ref[...], (((1,), (1,)), ((), ())),
        preferred_element_type=jnp.float32) + bqkv_ref[...]


_BQ = 256
_NQP = 10240


def _attn_body(qkv_blk, qkv_full, Wo_ref, bo_ref, L1w_ref, L1b_ref,
               L2w_ref, L2b_ref, L3w_ref, L3b_ref, out_ref):
    scale = 1.0 / np.sqrt(_HID // _NH)
    qb = qkv_blk[...][:, 0:_HID] * scale
    kv = qkv_full[...]
    kf = kv[:, _HID:2 * _HID]
    vf = kv[:, 2 * _HID:3 * _HID]
    outs = []
    for hh in range(_NH):
        lo = 8 * hh
        qh = qb[:, lo:lo + 8]
        kh = kf[:, lo:lo + 8]
        vh = vf[:, lo:lo + 8]
        sc = lax.dot_general(qh, kh, (((1,), (1,)), ((), ())),
                             preferred_element_type=jnp.float32)
        m = jnp.max(sc, axis=1, keepdims=True)
        e = jnp.exp(sc - m)
        s = jnp.sum(e, axis=1, keepdims=True)
        outs.append(lax.dot_general(e, vh, (((1,), (0,)), ((), ())),
                                    preferred_element_type=jnp.float32) / s)
    o = jnp.concatenate(outs, axis=1)
    h = lax.dot_general(o, Wo_ref[...], (((1,), (1,)), ((), ())),
                        preferred_element_type=jnp.float32) + bo_ref[...]
    h = jnp.maximum(lax.dot_general(h, L1w_ref[...], (((1,), (1,)), ((), ())),
                                    preferred_element_type=jnp.float32)
                    + L1b_ref[...], 0.0)
    h = jnp.maximum(lax.dot_general(h, L2w_ref[...], (((1,), (1,)), ((), ())),
                                    preferred_element_type=jnp.float32)
                    + L2b_ref[...], 0.0)
    out_ref[...] = jnp.sum(h * L3w_ref[...], axis=1, keepdims=True) \
        + L3b_ref[0, 0]


def _full(shape):
    return pl.BlockSpec(shape, lambda i: (0,) * len(shape))


def kernel(x, edge_index, params):
    p = params
    ei = edge_index.astype(jnp.int32)
    pad = _NW * _EPTP - _E
    e_src = jnp.concatenate(
        [ei[0], jnp.zeros((pad,), jnp.int32)]).reshape(_NW, _NCHUNK, _CHUNK)
    e_dst = jnp.concatenate(
        [ei[1], jnp.full((pad,), _DUMP, jnp.int32)]).reshape(
            _NW, _NCHUNK, _CHUNK)
    x16 = jnp.pad(x, ((0, 0), (0, 16 - x.shape[1])))
    W0p = jnp.pad(p['W0'], ((0, 16 - p['W0'].shape[0]), (0, 0)))
    zeros16 = jnp.zeros((_NP, 16), jnp.float32)
    zeros64 = jnp.zeros((_NP, _HID), jnp.float32)
    ones_chunk = jnp.ones((_CHUNK, 16), jnp.float32)

    def r2(v):
        return v.reshape(1, -1)

    deg2 = _sc_degree(e_dst, ones_chunk, zeros16)
    xp, dis = pl.pallas_call(
        _prep_body,
        out_shape=(jax.ShapeDtypeStruct((_N, 16), jnp.float32),
                   jax.ShapeDtypeStruct((_N, 1), jnp.float32)),
    )(deg2, x16)

    aggx = _sc_scatter16(xp, e_src, e_dst, zeros16)
    hp1 = pl.pallas_call(
        _layer0_body,
        out_shape=jax.ShapeDtypeStruct((_N, _HID), jnp.float32),
    )(aggx, xp, dis, W0p, r2(p['b0']), r2(p['g0']), r2(p['be0']), p['W1'])

    agg1 = _sc_scatter64(hp1, e_src, e_dst, zeros64)
    hp2 = pl.pallas_call(
        _layer_body,
        out_shape=jax.ShapeDtypeStruct((_N, _HID), jnp.float32),
    )(agg1, hp1, dis, r2(p['b1']), r2(p['g1']), r2(p['be1']), p['W2'])

    agg2 = _sc_scatter64(hp2, e_src, e_dst, zeros64)
    qkv = pl.pallas_call(
        _qkv_body,
        out_shape=jax.ShapeDtypeStruct((_N, 3 * _HID), jnp.float32),
    )(agg2, hp2, dis, r2(p['b2']), r2(p['g2']), r2(p['be2']), p['Wqkv'],
      r2(p['bqkv']))

    qkvp = jnp.pad(qkv, ((0, _NQP - _N), (0, 0)))
    out = pl.pallas_call(
        _attn_body,
        grid=(_NQP // _BQ,),
        in_specs=[
            pl.BlockSpec((_BQ, 3 * _HID), lambda i: (i, 0)),
            _full((_N, 3 * _HID)),
            _full((_HID, _HID)),
            _full((1, _HID)),
            _full((_HID // 2, _HID)),
            _full((1, _HID // 2)),
            _full((_HID // 4, _HID // 2)),
            _full((1, _HID // 4)),
            _full((1, _HID // 4)),
            _full((1, 1)),
        ],
        out_specs=pl.BlockSpec((_BQ, 1), lambda i: (i, 0)),
        out_shape=jax.ShapeDtypeStruct((_NQP, 1), jnp.float32),
    )(qkvp, qkv, p['Wo'], r2(p['bo']), p['L1w'], r2(p['L1b']), p['L2w'],
      r2(p['L2b']), p['L3w'], r2(p['L3b']))
    return out[:_N, 0]


# speedup vs baseline: 15.7074x; 1.0297x over previous
import functools

import jax
import jax.numpy as jnp
import numpy as np
from jax import lax
from jax.experimental import pallas as pl
from jax.experimental.pallas import tpu as pltpu
from jax.experimental.pallas import tpu_sc as plsc

_N = 10000
_E = 640000
_HID = 64
_NH = 8
_NC = 2
_NS = 16
_NW = _NC * _NS
_EPT = _E // _NW
_CHUNK = 128
_NCHUNK = -(-_EPT // _CHUNK)
_EPTP = _NCHUNK * _CHUNK
_NP = 10112
_RPT = _NP // _NS
_DUMP = _N + 64
_NBUF = 5

_mesh = plsc.VectorSubcoreMesh(core_axis_name="c", subcore_axis_name="s")


@functools.partial(
    pl.kernel,
    mesh=_mesh,
    compiler_params=pltpu.CompilerParams(use_tc_tiling_on_sc=False),
    out_type=jax.ShapeDtypeStruct((_NC, _NP, 16), jnp.float32),
    scratch_types=[
        pltpu.VMEM((_NCHUNK, _CHUNK), jnp.int32),
        pltpu.VMEM((_CHUNK, 16), jnp.float32),
        pltpu.VMEM_SHARED((_NP, 16), jnp.float32),
    ],
)
def _sc_degree(dst3_hbm, ones_hbm, zeros_hbm, out_hbm, dst_all, ones_v, deg_s):
    c = lax.axis_index("c")
    s = lax.axis_index("s")
    wid = c * _NS + s
    row0 = pl.multiple_of(s * _RPT, 8)
    pltpu.sync_copy(zeros_hbm.at[pl.ds(row0, _RPT)],
                    deg_s.at[pl.ds(row0, _RPT)])
    pltpu.sync_copy(ones_hbm, ones_v)
    pltpu.sync_copy(dst3_hbm.at[wid], dst_all)
    plsc.subcore_barrier()

    def body(j, carry):
        pltpu.sync_copy(ones_v, deg_s.at[dst_all.at[j]], add=True)
        return carry

    lax.fori_loop(0, _NCHUNK, body, 0)
    plsc.subcore_barrier()
    pltpu.sync_copy(deg_s.at[pl.ds(row0, _RPT)],
                    out_hbm.at[c, pl.ds(row0, _RPT)])


def _make_edge_scatter(width, nphase):
    hw = width // nphase

    @functools.partial(
        pl.kernel,
        mesh=_mesh,
        compiler_params=pltpu.CompilerParams(use_tc_tiling_on_sc=False),
        out_type=jax.ShapeDtypeStruct((_NC, _NP, width), jnp.float32),
        scratch_types=[
            pltpu.VMEM((_NCHUNK, _CHUNK), jnp.int32),
            pltpu.VMEM((_NCHUNK, _CHUNK), jnp.int32),
            pltpu.VMEM((_NBUF, _CHUNK, hw), jnp.float32),
            pltpu.VMEM_SHARED((_NP, hw), jnp.float32),
        ] + [pltpu.VMEM_SHARED((_NP, hw), jnp.float32)] * nphase + [
            [pltpu.SemaphoreType.DMA] * _NBUF,
        ],
    )
    def scat(tab_hbm, src3_hbm, dst3_hbm, zeros_hbm, out_hbm, src_all,
             dst_all, rows, tab_s, *rest):
        aggs = rest[:nphase]
        gsems = rest[nphase]
        c = lax.axis_index("c")
        s = lax.axis_index("s")
        wid = c * _NS + s
        row0 = pl.multiple_of(s * _RPT, 8)
        pltpu.sync_copy(src3_hbm.at[wid], src_all)
        pltpu.sync_copy(dst3_hbm.at[wid], dst_all)
        for ph in range(nphase):
            pltpu.sync_copy(zeros_hbm.at[pl.ds(row0, _RPT)],
                            aggs[ph].at[pl.ds(row0, _RPT)])

        for ph in range(nphase):
            agg_s = aggs[ph]
            col0 = ph * hw
            pltpu.sync_copy(tab_hbm.at[pl.ds(row0, _RPT), pl.ds(col0, hw)],
                            tab_s.at[pl.ds(row0, _RPT)])
            plsc.subcore_barrier()

            for b in range(_NBUF):
                pltpu.async_copy(tab_s.at[src_all.at[b]], rows.at[b],
                                 gsems[b])

            def body(j, carry):
                for b in range(_NBUF):

                    @pl.when(j % _NBUF == b)
                    def _():
                        pltpu.make_async_copy(
                            tab_hbm.at[pl.ds(0, _CHUNK), pl.ds(0, hw)],
                            rows.at[b], gsems[b]).wait()
                        pltpu.sync_copy(rows.at[b], agg_s.at[dst_all.at[j]],
                                        add=True)

                        @pl.when(j + _NBUF < _NCHUNK)
                        def _():
                            pltpu.async_copy(
                                tab_s.at[src_all.at[j + _NBUF]], rows.at[b],
                                gsems[b])

                return carry

            lax.fori_loop(0, _NCHUNK, body, 0)
            plsc.subcore_barrier()
            pltpu.sync_copy(
                agg_s.at[pl.ds(row0, _RPT)],
                out_hbm.at[c, pl.ds(row0, _RPT), pl.ds(col0, hw)])

    return scat


_sc_scatter16 = _make_edge_scatter(16, 1)
_sc_scatter64 = _make_edge_scatter(_HID, 2)


def _prep_body(deg2_ref, x16_ref, xp_ref, dis_ref):
    d = deg2_ref[...]
    deg = d[0, :_N, 0:1] + d[1, :_N, 0:1] + 1.0
    dis = 1.0 / jnp.sqrt(deg)
    dis_ref[...] = dis
    xp_ref[...] = x16_ref[...] * dis


def _bn_relu(t, g, be):
    m = jnp.mean(t, axis=0, keepdims=True)
    v = jnp.mean((t - m) ** 2, axis=0, keepdims=True)
    h = (t - m) * (1.0 / jnp.sqrt(v + 1e-5)) * g + be
    return jnp.maximum(h, 0.0)


def _layer0_body(agg_ref, xp_ref, dis_ref, W0_ref, b_ref, g_ref, be_ref,
                 W1_ref, out_ref):
    a = agg_ref[...]
    dis = dis_ref[...]
    t16 = a[0, :_N] + a[1, :_N] + xp_ref[...]
    t = dis * lax.dot_general(t16, W0_ref[...], (((1,), (0,)), ((), ())),
                              preferred_element_type=jnp.float32) + b_ref[...]
    h = _bn_relu(t, g_ref[...], be_ref[...])
    out_ref[...] = dis * lax.dot_general(
        h, W1_ref[...], (((1,), (0,)), ((), ())),
        preferred_element_type=jnp.float32)


def _layer_body(agg_ref, hp_ref, dis_ref, b_ref, g_ref, be_ref, W_ref,
                out_ref):
    a = agg_ref[...]
    dis = dis_ref[...]
    t = dis * (a[0, :_N] + a[1, :_N] + hp_ref[...]) + b_ref[...]
    h = _bn_relu(t, g_ref[...], be_ref[...])
    out_ref[...] = dis * lax.dot_general(
        h, W_ref[...], (((1,), (0,)), ((), ())),
        preferred_element_type=jnp.float32)


def _qkv_body(agg_ref, hp_ref, dis_ref, b_ref, g_ref, be_ref, Wqkv_ref,
              bqkv_ref, out_ref):
    a = agg_ref[...]
    t = dis_ref[...] * (a[0, :_N] + a[1, :_N] + hp_ref[...]) + b_ref[...]
    h = _bn_relu(t, g_ref[...], be_ref[...])
    out_ref[...] = lax.dot_general(
        h, Wqkv_ref[...], (((1,), (1,)), ((), ())),
        preferred_element_type=jnp.float32) + bqkv_ref[...]


_BQ = 256
_NQP = 10240


def _attn_body(qkv_blk, qkv_full, Wo_ref, bo_ref, L1w_ref, L1b_ref,
               L2w_ref, L2b_ref, L3w_ref, L3b_ref, out_ref):
    scale = 1.0 / np.sqrt(_HID // _NH)
    qb = qkv_blk[...][:, 0:_HID] * scale
    kv = qkv_full[...]
    kf = kv[:, _HID:2 * _HID]
    vf = kv[:, 2 * _HID:3 * _HID]
    outs = []
    for hh in range(_NH):
        lo = 8 * hh
        qh = qb[:, lo:lo + 8]
        kh = kf[:, lo:lo + 8]
        vh = vf[:, lo:lo + 8]
        sc = lax.dot_general(qh, kh, (((1,), (1,)), ((), ())),
                             preferred_element_type=jnp.float32)
        m = jnp.max(sc, axis=1, keepdims=True)
        e = jnp.exp(sc - m)
        s = jnp.sum(e, axis=1, keepdims=True)
        outs.append(lax.dot_general(e, vh, (((1,), (0,)), ((), ())),
                                    preferred_element_type=jnp.float32) / s)
    o = jnp.concatenate(outs, axis=1)
    h = lax.dot_general(o, Wo_ref[...], (((1,), (1,)), ((), ())),
                        preferred_element_type=jnp.float32) + bo_ref[...]
    h = jnp.maximum(lax.dot_general(h, L1w_ref[...], (((1,), (1,)), ((), ())),
                                    preferred_element_type=jnp.float32)
                    + L1b_ref[...], 0.0)
    h = jnp.maximum(lax.dot_general(h, L2w_ref[...], (((1,), (1,)), ((), ())),
                                    preferred_element_type=jnp.float32)
                    + L2b_ref[...], 0.0)
    out_ref[...] = jnp.sum(h * L3w_ref[...], axis=1, keepdims=True) \
        + L3b_ref[0, 0]


def _full(shape):
    return pl.BlockSpec(shape, lambda i: (0,) * len(shape))


def kernel(x, edge_index, params):
    p = params
    ei = edge_index.astype(jnp.int32)
    pad = _NW * _EPTP - _E
    e_src = jnp.concatenate(
        [ei[0], jnp.zeros((pad,), jnp.int32)]).reshape(_NW, _NCHUNK, _CHUNK)
    e_dst = jnp.concatenate(
        [ei[1], jnp.full((pad,), _DUMP, jnp.int32)]).reshape(
            _NW, _NCHUNK, _CHUNK)
    x16 = jnp.pad(x, ((0, 0), (0, 16 - x.shape[1])))
    W0p = jnp.pad(p['W0'], ((0, 16 - p['W0'].shape[0]), (0, 0)))
    zeros16 = jnp.zeros((_NP, 16), jnp.float32)
    zeros32 = jnp.zeros((_NP, 32), jnp.float32)
    ones_chunk = jnp.ones((_CHUNK, 16), jnp.float32)

    def r2(v):
        return v.reshape(1, -1)

    deg2 = _sc_degree(e_dst, ones_chunk, zeros16)
    xp, dis = pl.pallas_call(
        _prep_body,
        out_shape=(jax.ShapeDtypeStruct((_N, 16), jnp.float32),
                   jax.ShapeDtypeStruct((_N, 1), jnp.float32)),
    )(deg2, x16)

    def padn(v):
        return jnp.pad(v, ((0, _NP - _N), (0, 0)))

    aggx = _sc_scatter16(padn(xp), e_src, e_dst, zeros16)
    hp1 = pl.pallas_call(
        _layer0_body,
        out_shape=jax.ShapeDtypeStruct((_N, _HID), jnp.float32),
    )(aggx, xp, dis, W0p, r2(p['b0']), r2(p['g0']), r2(p['be0']), p['W1'])

    agg1 = _sc_scatter64(padn(hp1), e_src, e_dst, zeros32)
    hp2 = pl.pallas_call(
        _layer_body,
        out_shape=jax.ShapeDtypeStruct((_N, _HID), jnp.float32),
    )(agg1, hp1, dis, r2(p['b1']), r2(p['g1']), r2(p['be1']), p['W2'])

    agg2 = _sc_scatter64(padn(hp2), e_src, e_dst, zeros32)
    qkv = pl.pallas_call(
        _qkv_body,
        out_shape=jax.ShapeDtypeStruct((_N, 3 * _HID), jnp.float32),
    )(agg2, hp2, dis, r2(p['b2']), r2(p['g2']), r2(p['be2']), p['Wqkv'],
      r2(p['bqkv']))

    qkvp = jnp.pad(qkv, ((0, _NQP - _N), (0, 0)))
    out = pl.pallas_call(
        _attn_body,
        grid=(_NQP // _BQ,),
        in_specs=[
            pl.BlockSpec((_BQ, 3 * _HID), lambda i: (i, 0)),
            _full((_N, 3 * _HID)),
            _full((_HID, _HID)),
            _full((1, _HID)),
            _full((_HID // 2, _HID)),
            _full((1, _HID // 2)),
            _full((_HID // 4, _HID // 2)),
            _full((1, _HID // 4)),
            _full((1, _HID // 4)),
            _full((1, 1)),
        ],
        out_specs=pl.BlockSpec((_BQ, 1), lambda i: (i, 0)),
        out_shape=jax.ShapeDtypeStruct((_NQP, 1), jnp.float32),
    )(qkvp, qkv, p['Wo'], r2(p['bo']), p['L1w'], r2(p['L1b']), p['L2w'],
      r2(p['L2b']), p['L3w'], r2(p['L3b']))
    return out[:_N, 0]
